# initial kernel scaffold (unmeasured)
import jax
import jax.numpy as jnp
from jax import lax
from jax.experimental import pallas as pl
from jax.experimental.pallas import tpu as pltpu

N_DEV = 4


def kernel(x, w_mat):
    m_per, k = x.shape
    _, n_per = w_mat.shape

    x = x.astype(jnp.bfloat16)
    w_mat = w_mat.astype(jnp.bfloat16)

    def body(x_ref, w_ref, out_ref, comm_ref, acc_ref, amax_ref,
             send_sems, recv_sems, store_sems, load_sems,
             amax_send_sems, amax_recv_sems):
        my = lax.axis_index("i")
        left = (my + N_DEV - 1) % N_DEV
        right = (my + 1) % N_DEV

        barrier_sem = pltpu.get_barrier_semaphore()
        for nbr in [left, right]:
            pl.semaphore_signal(
                barrier_sem, inc=1,
                device_id=(nbr,), device_id_type=pl.DeviceIdType.MESH,
            )
        pl.semaphore_wait(barrier_sem, 2)

        chunk_maxes = []

        def gemm_store(src_chunk_ref, origin, slot):
            block = jnp.dot(src_chunk_ref[...], w_ref[...],
                            preferred_element_type=jnp.float32)
            acc_ref[slot] = block
            chunk_maxes.append(jnp.max(jnp.abs(block)))
            cp = pltpu.make_async_copy(
                acc_ref.at[slot],
                out_ref.at[pl.ds(origin * m_per, m_per), :],
                store_sems.at[slot],
            )
            cp.start()
            cp.wait()

        gemm_store(x_ref, my, 0)

        for h in range(N_DEV - 1):
            src = x_ref if h == 0 else comm_ref.at[(h - 1) % 2]
            recv_slot = h % 2
            rdma = pltpu.make_async_remote_copy(
                src_ref=src,
                dst_ref=comm_ref.at[recv_slot],
                send_sem=send_sems.at[h % 2],
                recv_sem=recv_sems.at[recv_slot],
                device_id=(right,),
                device_id_type=pl.DeviceIdType.MESH,
            )
            rdma.start()
            rdma.wait()
            origin = (my + N_DEV - h - 1) % N_DEV
            gemm_store(comm_ref.at[recv_slot], origin, (h + 1) % 2)

        local_max = chunk_maxes[0]
        for m in chunk_maxes[1:]:
            local_max = jnp.maximum(local_max, m)
        amax_ref[pl.ds(my, 1)] = jnp.full((1, 8, 128), local_max,
                                          dtype=jnp.float32)

        sends = []
        for d in range(1, N_DEV):
            peer = (my + d) % N_DEV
            snd = pltpu.make_async_remote_copy(
                src_ref=amax_ref.at[my],
                dst_ref=amax_ref.at[my],
                send_sem=amax_send_sems.at[d],
                recv_sem=amax_recv_sems.at[my],
                device_id=(peer,),
                device_id_type=pl.DeviceIdType.MESH,
            )
            snd.start()
            sends.append(snd)
        for d in range(1, N_DEV):
            peer = (my + d) % N_DEV
            rcv = pltpu.make_async_remote_copy(
                src_ref=amax_ref.at[peer],
                dst_ref=amax_ref.at[peer],
                send_sem=amax_send_sems.at[0],
                recv_sem=amax_recv_sems.at[peer],
                device_id=(peer,),
                device_id_type=pl.DeviceIdType.MESH,
            )
            rcv.wait_recv()
        for snd in sends:
            snd.wait_send()

        gmax = jnp.max(amax_ref[...])
        scale = gmax / 448.0

        for b in range(N_DEV):
            slot = b % 2
            ld = pltpu.make_async_copy(
                out_ref.at[pl.ds(b * m_per, m_per), :],
                acc_ref.at[slot],
                load_sems.at[slot],
            )
            ld.start()
            ld.wait()
            y = acc_ref[slot]
            q = jnp.clip(y / scale, -448.0, 448.0)
            q = q.astype(jnp.float8_e4m3fn).astype(jnp.float32)
            acc_ref[slot] = q * scale
            st = pltpu.make_async_copy(
                acc_ref.at[slot],
                out_ref.at[pl.ds(b * m_per, m_per), :],
                store_sems.at[slot],
            )
            st.start()
            st.wait()

        def _exit(second_barrier):
            for nbr in [left, right]:
                pl.semaphore_signal(
                    second_barrier, inc=1,
                    device_id=(nbr,), device_id_type=pl.DeviceIdType.MESH,
                )
            pl.semaphore_wait(second_barrier, 2)

        pl.run_scoped(_exit, second_barrier=pltpu.SemaphoreType.REGULAR)

    return pl.pallas_call(
        body,
        out_shape=jax.ShapeDtypeStruct((N_DEV * m_per, n_per), jnp.float32),
        in_specs=[
            pl.BlockSpec(memory_space=pltpu.VMEM),
            pl.BlockSpec(memory_space=pltpu.VMEM),
        ],
        out_specs=pl.BlockSpec(memory_space=pltpu.ANY),
        scratch_shapes=[
            pltpu.VMEM((2, m_per, k), jnp.bfloat16),
            pltpu.VMEM((2, m_per, n_per), jnp.float32),
            pltpu.VMEM((N_DEV, 8, 128), jnp.float32),
            pltpu.SemaphoreType.DMA((2,)),
            pltpu.SemaphoreType.DMA((2,)),
            pltpu.SemaphoreType.DMA((2,)),
            pltpu.SemaphoreType.DMA((2,)),
            pltpu.SemaphoreType.DMA((N_DEV,)),
            pltpu.SemaphoreType.DMA((N_DEV,)),
        ],
        compiler_params=pltpu.CompilerParams(collective_id=0),
    )(x, w_mat)


# baseline (device time: 458555 ns/iter reference)
import jax
import jax.numpy as jnp
from jax import lax
from jax.experimental import pallas as pl
from jax.experimental.pallas import tpu as pltpu

N_DEV = 4


def kernel(x, w_mat):
    m_per, k = x.shape
    _, n_per = w_mat.shape

    x = x.astype(jnp.bfloat16)
    w_mat = w_mat.astype(jnp.bfloat16)

    def body(x_ref, w_ref, out_ref, comm_ref, acc_ref, amax_ref,
             send_sems, recv_sems, store_sems, load_sems,
             amax_send_sems, amax_recv_sems):
        my = lax.axis_index("i")
        left = (my + N_DEV - 1) % N_DEV
        right = (my + 1) % N_DEV

        barrier_sem = pltpu.get_barrier_semaphore()
        for nbr in [left, right]:
            pl.semaphore_signal(
                barrier_sem, inc=1,
                device_id=(nbr,), device_id_type=pl.DeviceIdType.MESH,
            )
        pl.semaphore_wait(barrier_sem, 2)

        chunk_maxes = []

        def gemm_store(src_chunk_ref, origin, slot):
            block = jnp.dot(src_chunk_ref[...], w_ref[...],
                            preferred_element_type=jnp.float32)
            acc_ref[slot] = block
            chunk_maxes.append(jnp.max(jnp.abs(block)))
            cp = pltpu.make_async_copy(
                acc_ref.at[slot],
                out_ref.at[pl.ds(origin * m_per, m_per), :],
                store_sems.at[slot],
            )
            cp.start()
            cp.wait()

        gemm_store(x_ref, my, 0)

        for h in range(N_DEV - 1):
            src = x_ref if h == 0 else comm_ref.at[(h - 1) % 2]
            recv_slot = h % 2
            rdma = pltpu.make_async_remote_copy(
                src_ref=src,
                dst_ref=comm_ref.at[recv_slot],
                send_sem=send_sems.at[h % 2],
                recv_sem=recv_sems.at[recv_slot],
                device_id=(right,),
                device_id_type=pl.DeviceIdType.MESH,
            )
            rdma.start()
            rdma.wait()
            origin = (my + N_DEV - h - 1) % N_DEV
            gemm_store(comm_ref.at[recv_slot], origin, 0)

        local_max = chunk_maxes[0]
        for m in chunk_maxes[1:]:
            local_max = jnp.maximum(local_max, m)
        amax_ref[pl.ds(my, 1)] = jnp.full((1, 8, 128), local_max,
                                          dtype=jnp.float32)

        sends = []
        for d in range(1, N_DEV):
            peer = (my + d) % N_DEV
            snd = pltpu.make_async_remote_copy(
                src_ref=amax_ref.at[my],
                dst_ref=amax_ref.at[my],
                send_sem=amax_send_sems.at[d],
                recv_sem=amax_recv_sems.at[my],
                device_id=(peer,),
                device_id_type=pl.DeviceIdType.MESH,
            )
            snd.start()
            sends.append(snd)
        for d in range(1, N_DEV):
            peer = (my + d) % N_DEV
            rcv = pltpu.make_async_remote_copy(
                src_ref=amax_ref.at[peer],
                dst_ref=amax_ref.at[peer],
                send_sem=amax_send_sems.at[0],
                recv_sem=amax_recv_sems.at[peer],
                device_id=(peer,),
                device_id_type=pl.DeviceIdType.MESH,
            )
            rcv.wait_recv()
        for snd in sends:
            snd.wait_send()

        gmax = jnp.max(amax_ref[...])
        scale = gmax / 448.0

        for b in range(N_DEV):
            slot = 0
            ld = pltpu.make_async_copy(
                out_ref.at[pl.ds(b * m_per, m_per), :],
                acc_ref.at[slot],
                load_sems.at[slot],
            )
            ld.start()
            ld.wait()
            y = acc_ref[slot]
            q = jnp.clip(y / scale, -448.0, 448.0)
            q = q.astype(jnp.float8_e4m3fn).astype(jnp.float32)
            acc_ref[slot] = q * scale
            st = pltpu.make_async_copy(
                acc_ref.at[slot],
                out_ref.at[pl.ds(b * m_per, m_per), :],
                store_sems.at[slot],
            )
            st.start()
            st.wait()

        def _exit(second_barrier):
            for nbr in [left, right]:
                pl.semaphore_signal(
                    second_barrier, inc=1,
                    device_id=(nbr,), device_id_type=pl.DeviceIdType.MESH,
                )
            pl.semaphore_wait(second_barrier, 2)

        pl.run_scoped(_exit, second_barrier=pltpu.SemaphoreType.REGULAR)

    return pl.pallas_call(
        body,
        out_shape=jax.ShapeDtypeStruct((N_DEV * m_per, n_per), jnp.float32),
        in_specs=[
            pl.BlockSpec(memory_space=pltpu.VMEM),
            pl.BlockSpec(memory_space=pltpu.VMEM),
        ],
        out_specs=pl.BlockSpec(memory_space=pl.ANY),
        scratch_shapes=[
            pltpu.VMEM((2, m_per, k), jnp.bfloat16),
            pltpu.VMEM((1, m_per, n_per), jnp.float32),
            pltpu.VMEM((N_DEV, 8, 128), jnp.float32),
            pltpu.SemaphoreType.DMA((2,)),
            pltpu.SemaphoreType.DMA((2,)),
            pltpu.SemaphoreType.DMA((2,)),
            pltpu.SemaphoreType.DMA((2,)),
            pltpu.SemaphoreType.DMA((N_DEV,)),
            pltpu.SemaphoreType.DMA((N_DEV,)),
        ],
        compiler_params=pltpu.CompilerParams(
            collective_id=0,
            vmem_limit_bytes=60 * 1024 * 1024,
        ),
    )(x, w_mat)
